# baseline (speedup 1.0000x reference)
import functools

import jax
import jax.numpy as jnp
from jax import lax
from jax.experimental import pallas as pl
from jax.experimental.pallas import tpu as pltpu
from jax.experimental.pallas import tpu_sc as plsc

_N = 10000
_F = 128
_NCLASS = 40
_DPAD = 64
_E = 320000
_CHUNK = 128
_NC = 2
_NS = 16
_NW = _NC * _NS
_CPT = 80
_EPAD = _NW * _CHUNK * _CPT
_CPTI = _CPT + 2
_NPAD = 10240
_RPT = _NPAD // _NS


def _make_sc_pass(D):
    mesh = plsc.VectorSubcoreMesh(core_axis_name="c", subcore_axis_name="s")

    @functools.partial(
        pl.kernel,
        mesh=mesh,
        compiler_params=pltpu.CompilerParams(use_tc_tiling_on_sc=False),
        out_type=jax.ShapeDtypeStruct((_NC, _NPAD, D), jnp.float32),
        scratch_types=[
            pltpu.VMEM_SHARED((_NPAD, D), jnp.float32),
            pltpu.VMEM((_CPTI, _CHUNK), jnp.int32),
            pltpu.VMEM((4, _CHUNK), jnp.int32),
            pltpu.VMEM((4, _CHUNK), jnp.float32),
            pltpu.VMEM((_CHUNK, D), jnp.float32),
            pltpu.VMEM((_CHUNK, D), jnp.float32),
            pltpu.SemaphoreType.DMA,
            pltpu.SemaphoreType.DMA,
            pltpu.SemaphoreType.DMA,
            pltpu.SemaphoreType.DMA,
            pltpu.SemaphoreType.DMA,
        ],
    )
    def sc_pass(h_hbm, src_hbm, dst_hbm, ew_hbm, zero_hbm, out_hbm,
                acc, srcv, dstb, ewb, rows_a, rows_b,
                sem_ga, sem_gb, sem_sa, sem_sb, sem_i):
        c = lax.axis_index("c")
        s = lax.axis_index("s")
        wid = s * _NC + c
        pltpu.sync_copy(src_hbm.at[wid], srcv)
        pltpu.sync_copy(dst_hbm.at[wid, pl.ds(0, 2)], dstb.at[pl.ds(0, 2)])
        pltpu.sync_copy(ew_hbm.at[wid, pl.ds(0, 2)], ewb.at[pl.ds(0, 2)])
        pltpu.sync_copy(zero_hbm, acc.at[pl.ds(s * _RPT, _RPT)])
        plsc.subcore_barrier()

        def scale(rows, j):
            def grp_body(g, carry2):
                ewg = ewb[j, pl.ds(g * 16, 16)]
                for l in range(16):
                    wvec = lax.gather(
                        ewg, jnp.full((16, 1), l, jnp.int32),
                        lax.GatherDimensionNumbers(
                            offset_dims=(), collapsed_slice_dims=(0,),
                            start_index_map=(0,)),
                        slice_sizes=(1,),
                        mode=lax.GatherScatterMode.PROMISE_IN_BOUNDS)
                    r = g * 16 + l
                    for f in range(D // 16):
                        sl = pl.ds(f * 16, 16)
                        rows[r, sl] = rows[r, sl] * wvec
                return carry2

            lax.fori_loop(0, _CHUNK // 16, grp_body, 0)

        def wait_gather(rows, j, sem):
            pltpu.make_async_copy(h_hbm.at[srcv.at[j]], rows, sem).wait()

        def wait_scatter(rows, j, sem):
            pltpu.make_async_copy(rows, acc.at[dstb.at[j]], sem).wait()

        pltpu.async_copy(h_hbm.at[srcv.at[0]], rows_a, sem_ga)
        pltpu.async_copy(h_hbm.at[srcv.at[1]], rows_b, sem_gb)

        def super_body(k, carry):
            ja = 2 * k
            jb = 2 * k + 1
            off = (k % 2) * 2
            noff = 2 - off
            nj = jnp.minimum(ja + 2, _CPT - 2)
            pltpu.async_copy(dst_hbm.at[wid, pl.ds(nj, 2)],
                             dstb.at[pl.ds(noff, 2)], sem_i)
            pltpu.async_copy(ew_hbm.at[wid, pl.ds(nj, 2)],
                             ewb.at[pl.ds(noff, 2)], sem_i)
            wait_gather(rows_a, ja, sem_ga)
            scale(rows_a, off)
            pltpu.async_copy(rows_a, acc.at[dstb.at[off]], sem_sa, add=True)
            wait_gather(rows_b, jb, sem_gb)
            scale(rows_b, off + 1)
            pltpu.async_copy(rows_b, acc.at[dstb.at[off + 1]], sem_sb, add=True)
            wait_scatter(rows_a, off, sem_sa)
            pltpu.async_copy(h_hbm.at[srcv.at[ja + 2]], rows_a, sem_ga)
            wait_scatter(rows_b, off + 1, sem_sb)
            pltpu.async_copy(h_hbm.at[srcv.at[jb + 2]], rows_b, sem_gb)
            pltpu.make_async_copy(dst_hbm.at[wid, pl.ds(nj, 2)],
                                  dstb.at[pl.ds(noff, 2)], sem_i).wait()
            pltpu.make_async_copy(ew_hbm.at[wid, pl.ds(nj, 2)],
                                  ewb.at[pl.ds(noff, 2)], sem_i).wait()
            return carry

        lax.fori_loop(0, _CPT // 2, super_body, 0)
        wait_gather(rows_a, _CPT, sem_ga)
        wait_gather(rows_b, _CPT + 1, sem_gb)
        plsc.subcore_barrier()
        pltpu.sync_copy(acc.at[pl.ds(s * _RPT, _RPT)],
                        out_hbm.at[c, pl.ds(s * _RPT, _RPT)])

    return sc_pass


_sc_pass_128 = _make_sc_pass(_F)
_sc_pass_64 = _make_sc_pass(_DPAD)


def _tc_matmul(x, w):
    def body(x_ref, w_ref, o_ref):
        o_ref[...] = jnp.dot(x_ref[...], w_ref[...],
                             preferred_element_type=jnp.float32)

    return pl.pallas_call(
        body,
        out_shape=jax.ShapeDtypeStruct((x.shape[0], w.shape[1]), jnp.float32),
    )(x, w)


def _tc_mid(parts, b1, w2p):
    def body(p_ref, b1_ref, w_ref, o_ref):
        t = p_ref[0, :_N] + p_ref[1, :_N] + b1_ref[...]
        cm = jnp.mean(t, axis=0, keepdims=True)
        xc = t - cm
        ms = jnp.sum(xc * xc) / _N
        inv = lax.rsqrt(ms + 1e-6)
        h = jnp.maximum(xc * inv, 0.0)
        o_ref[...] = jnp.dot(h, w_ref[...], preferred_element_type=jnp.float32)

    return pl.pallas_call(
        body,
        out_shape=jax.ShapeDtypeStruct((_N, _DPAD), jnp.float32),
    )(parts, b1.reshape(1, -1), w2p)


def _tc_final(parts, b2):
    def body(q_ref, b2_ref, o_ref):
        ssum = q_ref[0, :_N] + q_ref[1, :_N]
        o_ref[...] = ssum[:, :_NCLASS] + b2_ref[...]

    return pl.pallas_call(
        body,
        out_shape=jax.ShapeDtypeStruct((_N, _NCLASS), jnp.float32),
    )(parts, b2.reshape(1, -1))


def kernel(x, edge_index, edge_attr, W1, b1, W2, b2):
    src = edge_index[0].astype(jnp.int32)
    dst = edge_index[1].astype(jnp.int32)
    ew = edge_attr.astype(jnp.float32)
    pad = _EPAD - _E
    src2 = jnp.concatenate([src, jnp.zeros((pad,), jnp.int32)]
                           ).reshape(_NW, _CPT, _CHUNK)
    src2 = jnp.concatenate(
        [src2, jnp.zeros((_NW, _CPTI - _CPT, _CHUNK), jnp.int32)], axis=1)
    dst2 = jnp.concatenate([dst, jnp.zeros((pad,), jnp.int32)]
                           ).reshape(_NW, _CPT, _CHUNK)
    ew2 = jnp.concatenate([ew, jnp.zeros((pad,), jnp.float32)]
                          ).reshape(_NW, _CPT, _CHUNK)
    zeros_f = jnp.zeros((_RPT, _F), jnp.float32)
    zeros_d = jnp.zeros((_RPT, _DPAD), jnp.float32)
    w2p = jnp.pad(W2, ((0, 0), (0, _DPAD - _NCLASS)))

    h1 = _tc_matmul(x, W1)
    p1 = _sc_pass_128(h1, src2, dst2, ew2, zeros_f)
    p = _tc_mid(p1, b1, w2p)
    p2 = _sc_pass_64(p, src2, dst2, ew2, zeros_d)
    return _tc_final(p2, b2)


# speedup vs baseline: 1.2165x; 1.2165x over previous
import functools

import jax
import jax.numpy as jnp
from jax import lax
from jax.experimental import pallas as pl
from jax.experimental.pallas import tpu as pltpu
from jax.experimental.pallas import tpu_sc as plsc

_N = 10000
_F = 128
_NCLASS = 40
_DPAD = 64
_E = 320000
_CHUNK = 128
_NC = 2
_NS = 16
_NW = _NC * _NS
_CPT = 80
_EPAD = _NW * _CHUNK * _CPT
_CPTI = _CPT + 2
_NPAD = 10240
_RPT = _NPAD // _NS


def _make_sc_pass(D, pipelined):
    mesh = plsc.VectorSubcoreMesh(core_axis_name="c", subcore_axis_name="s")
    row_bufs = ([pltpu.VMEM((_CHUNK, D), jnp.float32)] * 2 if pipelined
                else [pltpu.VMEM((_CHUNK, D), jnp.float32)])
    sems = [pltpu.SemaphoreType.DMA] * (4 if pipelined else 1)

    @functools.partial(
        pl.kernel,
        mesh=mesh,
        compiler_params=pltpu.CompilerParams(use_tc_tiling_on_sc=False),
        out_type=jax.ShapeDtypeStruct((_NC, _NPAD, D), jnp.float32),
        scratch_types=[
            pltpu.VMEM_SHARED((_NPAD, D), jnp.float32),
            pltpu.VMEM((_CPTI, _CHUNK), jnp.int32),
            pltpu.VMEM((_CPT, _CHUNK), jnp.int32),
            pltpu.VMEM((_CPT, _CHUNK), jnp.float32),
        ] + row_bufs + sems,
    )
    def sc_pass(h_hbm, src_hbm, dst_hbm, ew_hbm, zero_hbm, out_hbm,
                acc, srcv, dstv, ewv, *bufs_and_sems):
        if pipelined:
            rows_a, rows_b, sem_ga, sem_gb, sem_sa, sem_sb = bufs_and_sems
        else:
            rows_a, sem_ga = bufs_and_sems
        c = lax.axis_index("c")
        s = lax.axis_index("s")
        wid = s * _NC + c
        pltpu.sync_copy(src_hbm.at[wid], srcv)
        pltpu.sync_copy(dst_hbm.at[wid], dstv)
        pltpu.sync_copy(ew_hbm.at[wid], ewv)
        pltpu.sync_copy(zero_hbm, acc.at[pl.ds(s * _RPT, _RPT)])
        plsc.subcore_barrier()

        def scale(rows, j):
            def grp_body(g, carry2):
                ewg = ewv[j, pl.ds(g * 16, 16)]
                for l in range(16):
                    wvec = lax.gather(
                        ewg, jnp.full((16, 1), l, jnp.int32),
                        lax.GatherDimensionNumbers(
                            offset_dims=(), collapsed_slice_dims=(0,),
                            start_index_map=(0,)),
                        slice_sizes=(1,),
                        mode=lax.GatherScatterMode.PROMISE_IN_BOUNDS)
                    r = g * 16 + l
                    for f in range(D // 16):
                        sl = pl.ds(f * 16, 16)
                        rows[r, sl] = rows[r, sl] * wvec
                return carry2

            lax.fori_loop(0, _CHUNK // 16, grp_body, 0)

        def wait_gather(rows, j, sem):
            pltpu.make_async_copy(h_hbm.at[srcv.at[j]], rows, sem).wait()

        def wait_scatter(rows, j, sem):
            pltpu.make_async_copy(rows, acc.at[dstv.at[j]], sem).wait()

        if pipelined:
            pltpu.async_copy(h_hbm.at[srcv.at[0]], rows_a, sem_ga)
            pltpu.async_copy(h_hbm.at[srcv.at[1]], rows_b, sem_gb)

            def super_body(k, carry):
                ja = 2 * k
                jb = 2 * k + 1
                wait_gather(rows_a, ja, sem_ga)
                scale(rows_a, ja)
                pltpu.async_copy(rows_a, acc.at[dstv.at[ja]], sem_sa, add=True)
                wait_gather(rows_b, jb, sem_gb)
                scale(rows_b, jb)
                pltpu.async_copy(rows_b, acc.at[dstv.at[jb]], sem_sb, add=True)
                wait_scatter(rows_a, ja, sem_sa)
                pltpu.async_copy(h_hbm.at[srcv.at[ja + 2]], rows_a, sem_ga)
                wait_scatter(rows_b, jb, sem_sb)
                pltpu.async_copy(h_hbm.at[srcv.at[jb + 2]], rows_b, sem_gb)
                return carry

            lax.fori_loop(0, _CPT // 2, super_body, 0)
            wait_gather(rows_a, _CPT, sem_ga)
            wait_gather(rows_b, _CPT + 1, sem_gb)
        else:
            def chunk_body(j, carry):
                pltpu.async_copy(h_hbm.at[srcv.at[j]], rows_a, sem_ga).wait()
                scale(rows_a, j)
                pltpu.sync_copy(rows_a, acc.at[dstv.at[j]], add=True)
                return carry

            lax.fori_loop(0, _CPT, chunk_body, 0)
        plsc.subcore_barrier()
        pltpu.sync_copy(acc.at[pl.ds(s * _RPT, _RPT)],
                        out_hbm.at[c, pl.ds(s * _RPT, _RPT)])

    return sc_pass


_sc_pass_128 = _make_sc_pass(_F, pipelined=False)
_sc_pass_64 = _make_sc_pass(_DPAD, pipelined=True)


def _tc_matmul(x, w):
    def body(x_ref, w_ref, o_ref):
        o_ref[...] = jnp.dot(x_ref[...], w_ref[...],
                             preferred_element_type=jnp.float32)

    return pl.pallas_call(
        body,
        out_shape=jax.ShapeDtypeStruct((x.shape[0], w.shape[1]), jnp.float32),
    )(x, w)


def _tc_mid(parts, b1, w2p):
    def body(p_ref, b1_ref, w_ref, o_ref):
        t = p_ref[0, :_N] + p_ref[1, :_N] + b1_ref[...]
        cm = jnp.mean(t, axis=0, keepdims=True)
        xc = t - cm
        ms = jnp.sum(xc * xc) / _N
        inv = lax.rsqrt(ms + 1e-6)
        h = jnp.maximum(xc * inv, 0.0)
        o_ref[...] = jnp.dot(h, w_ref[...], preferred_element_type=jnp.float32)

    return pl.pallas_call(
        body,
        out_shape=jax.ShapeDtypeStruct((_N, _DPAD), jnp.float32),
    )(parts, b1.reshape(1, -1), w2p)


def _tc_final(parts, b2):
    def body(q_ref, b2_ref, o_ref):
        ssum = q_ref[0, :_N] + q_ref[1, :_N]
        o_ref[...] = ssum[:, :_NCLASS] + b2_ref[...]

    return pl.pallas_call(
        body,
        out_shape=jax.ShapeDtypeStruct((_N, _NCLASS), jnp.float32),
    )(parts, b2.reshape(1, -1))


def kernel(x, edge_index, edge_attr, W1, b1, W2, b2):
    src = edge_index[0].astype(jnp.int32)
    dst = edge_index[1].astype(jnp.int32)
    ew = edge_attr.astype(jnp.float32)
    pad = _EPAD - _E
    src2 = jnp.concatenate([src, jnp.zeros((pad,), jnp.int32)]
                           ).reshape(_NW, _CPT, _CHUNK)
    src2 = jnp.concatenate(
        [src2, jnp.zeros((_NW, _CPTI - _CPT, _CHUNK), jnp.int32)], axis=1)
    dst2 = jnp.concatenate([dst, jnp.zeros((pad,), jnp.int32)]
                           ).reshape(_NW, _CPT, _CHUNK)
    ew2 = jnp.concatenate([ew, jnp.zeros((pad,), jnp.float32)]
                          ).reshape(_NW, _CPT, _CHUNK)
    zeros_f = jnp.zeros((_RPT, _F), jnp.float32)
    zeros_d = jnp.zeros((_RPT, _DPAD), jnp.float32)
    w2p = jnp.pad(W2, ((0, 0), (0, _DPAD - _NCLASS)))

    h1 = _tc_matmul(x, W1)
    p1 = _sc_pass_128(h1, src2, dst2, ew2, zeros_f)
    p = _tc_mid(p1, b1, w2p)
    p2 = _sc_pass_64(p, src2, dst2, ew2, zeros_d)
    return _tc_final(p2, b2)


# speedup vs baseline: 1.3041x; 1.0720x over previous
import functools

import jax
import jax.numpy as jnp
from jax import lax
from jax.experimental import pallas as pl
from jax.experimental.pallas import tpu as pltpu
from jax.experimental.pallas import tpu_sc as plsc

_N = 10000
_F = 128
_NCLASS = 40
_DPAD = 64
_E = 320000
_CHUNK = 128
_NC = 2
_NS = 16
_NW = _NC * _NS
_CPT = 80
_EPAD = _NW * _CHUNK * _CPT
_CPTI = _CPT + 2
_NPAD = 10240
_RPT = _NPAD // _NS


def _make_sc_pass(D, rpo):
    mesh = plsc.VectorSubcoreMesh(core_axis_name="c", subcore_axis_name="s")
    R = rpo // _CHUNK
    nops = _CPT // R
    idx_shape = (_CPT, _CHUNK) if R == 1 else (_CPT * _CHUNK,)

    @functools.partial(
        pl.kernel,
        mesh=mesh,
        compiler_params=pltpu.CompilerParams(use_tc_tiling_on_sc=False),
        out_type=jax.ShapeDtypeStruct((_NC, _NPAD, D), jnp.float32),
        scratch_types=[
            pltpu.VMEM_SHARED((_NPAD, D), jnp.float32),
            pltpu.VMEM(idx_shape, jnp.int32),
            pltpu.VMEM(idx_shape, jnp.int32),
            pltpu.VMEM((nops, rpo), jnp.float32),
            pltpu.VMEM((rpo, D), jnp.float32),
            pltpu.SemaphoreType.DMA,
        ],
    )
    def sc_pass(h_hbm, src_hbm, dst_hbm, ew_hbm, zero_hbm, out_hbm,
                acc, srcv, dstv, ewv, rows, sem):
        c = lax.axis_index("c")
        s = lax.axis_index("s")
        wid = s * _NC + c
        pltpu.sync_copy(src_hbm.at[wid], srcv)
        pltpu.sync_copy(dst_hbm.at[wid], dstv)
        pltpu.sync_copy(ew_hbm.at[wid], ewv)
        pltpu.sync_copy(zero_hbm, acc.at[pl.ds(s * _RPT, _RPT)])
        plsc.subcore_barrier()

        def src_idx(j):
            return srcv.at[j] if R == 1 else srcv.at[pl.ds(j * rpo, rpo)]

        def dst_idx(j):
            return dstv.at[j] if R == 1 else dstv.at[pl.ds(j * rpo, rpo)]

        def scale(j):
            def grp_body(g, carry2):
                ewg = ewv[j, pl.ds(g * 16, 16)]
                for l in range(16):
                    wvec = lax.gather(
                        ewg, jnp.full((16, 1), l, jnp.int32),
                        lax.GatherDimensionNumbers(
                            offset_dims=(), collapsed_slice_dims=(0,),
                            start_index_map=(0,)),
                        slice_sizes=(1,),
                        mode=lax.GatherScatterMode.PROMISE_IN_BOUNDS)
                    r = g * 16 + l
                    for f in range(D // 16):
                        sl = pl.ds(f * 16, 16)
                        rows[r, sl] = rows[r, sl] * wvec
                return carry2

            lax.fori_loop(0, rpo // 16, grp_body, 0)

        def chunk_body(j, carry):
            pltpu.async_copy(h_hbm.at[src_idx(j)], rows, sem).wait()
            scale(j)
            pltpu.sync_copy(rows, acc.at[dst_idx(j)], add=True)
            return carry

        lax.fori_loop(0, nops, chunk_body, 0)
        plsc.subcore_barrier()
        pltpu.sync_copy(acc.at[pl.ds(s * _RPT, _RPT)],
                        out_hbm.at[c, pl.ds(s * _RPT, _RPT)])

    return sc_pass


_sc_pass_128 = _make_sc_pass(_F, rpo=128)
_sc_pass_64 = _make_sc_pass(_DPAD, rpo=512)


def _tc_matmul(x, w):
    def body(x_ref, w_ref, o_ref):
        o_ref[...] = jnp.dot(x_ref[...], w_ref[...],
                             preferred_element_type=jnp.float32)

    return pl.pallas_call(
        body,
        out_shape=jax.ShapeDtypeStruct((x.shape[0], w.shape[1]), jnp.float32),
    )(x, w)


def _tc_mid(parts, b1, w2p):
    def body(p_ref, b1_ref, w_ref, o_ref):
        t = p_ref[0, :_N] + p_ref[1, :_N] + b1_ref[...]
        cm = jnp.mean(t, axis=0, keepdims=True)
        xc = t - cm
        ms = jnp.sum(xc * xc) / _N
        inv = lax.rsqrt(ms + 1e-6)
        h = jnp.maximum(xc * inv, 0.0)
        o_ref[...] = jnp.dot(h, w_ref[...], preferred_element_type=jnp.float32)

    return pl.pallas_call(
        body,
        out_shape=jax.ShapeDtypeStruct((_N, _DPAD), jnp.float32),
    )(parts, b1.reshape(1, -1), w2p)


def _tc_final(parts, b2):
    def body(q_ref, b2_ref, o_ref):
        ssum = q_ref[0, :_N] + q_ref[1, :_N]
        o_ref[...] = ssum[:, :_NCLASS] + b2_ref[...]

    return pl.pallas_call(
        body,
        out_shape=jax.ShapeDtypeStruct((_N, _NCLASS), jnp.float32),
    )(parts, b2.reshape(1, -1))


def kernel(x, edge_index, edge_attr, W1, b1, W2, b2):
    src = edge_index[0].astype(jnp.int32)
    dst = edge_index[1].astype(jnp.int32)
    ew = edge_attr.astype(jnp.float32)
    pad = _EPAD - _E
    src2 = jnp.concatenate([src, jnp.zeros((pad,), jnp.int32)]
                           ).reshape(_NW, _CPT, _CHUNK)
    dst2 = jnp.concatenate([dst, jnp.zeros((pad,), jnp.int32)]
                           ).reshape(_NW, _CPT, _CHUNK)
    ew2 = jnp.concatenate([ew, jnp.zeros((pad,), jnp.float32)]
                          ).reshape(_NW, _CPT, _CHUNK)
    ew2b = ew2.reshape(_NW, _CPT // 4, 4 * _CHUNK)
    src2f = src2.reshape(_NW, _CPT * _CHUNK)
    dst2f = dst2.reshape(_NW, _CPT * _CHUNK)
    zeros_f = jnp.zeros((_RPT, _F), jnp.float32)
    zeros_d = jnp.zeros((_RPT, _DPAD), jnp.float32)
    w2p = jnp.pad(W2, ((0, 0), (0, _DPAD - _NCLASS)))

    h1 = _tc_matmul(x, W1)
    p1 = _sc_pass_128(h1, src2, dst2, ew2, zeros_f)
    p = _tc_mid(p1, b1, w2p)
    p2 = _sc_pass_64(p, src2f, dst2f, ew2b, zeros_d)
    return _tc_final(p2, b2)


# speedup vs baseline: 1.4916x; 1.1438x over previous
import functools

import jax
import jax.numpy as jnp
from jax import lax
from jax.experimental import pallas as pl
from jax.experimental.pallas import tpu as pltpu
from jax.experimental.pallas import tpu_sc as plsc

_N = 10000
_F = 128
_NCLASS = 40
_DPAD = 64
_E = 320000
_CHUNK = 128
_NC = 2
_NS = 16
_NW = _NC * _NS
_CPT = 80
_EPAD = _NW * _CHUNK * _CPT
_CPTI = _CPT + 2
_NPAD = 10240
_RPT = _NPAD // _NS


def _make_sc_pass(D, table_in_spmem=False):
    mesh = plsc.VectorSubcoreMesh(core_axis_name="c", subcore_axis_name="s")
    tab_scratch = ([pltpu.VMEM_SHARED((_NPAD, D), jnp.float32)]
                   if table_in_spmem else [])

    @functools.partial(
        pl.kernel,
        mesh=mesh,
        compiler_params=pltpu.CompilerParams(use_tc_tiling_on_sc=False),
        out_type=jax.ShapeDtypeStruct((_NC, _NPAD, D), jnp.float32),
        scratch_types=[
            pltpu.VMEM_SHARED((_NPAD, D), jnp.float32),
            pltpu.VMEM((_CPT, _CHUNK), jnp.int32),
            pltpu.VMEM((_CPT, _CHUNK), jnp.int32),
            pltpu.VMEM((_CPT, _CHUNK), jnp.float32),
            pltpu.VMEM((_CHUNK, D), jnp.float32),
            pltpu.SemaphoreType.DMA,
        ] + tab_scratch,
    )
    def sc_pass(h_hbm, src_hbm, dst_hbm, ew_hbm, zero_hbm, out_hbm,
                acc, srcv, dstv, ewv, rows, sem, *tab):
        c = lax.axis_index("c")
        s = lax.axis_index("s")
        wid = s * _NC + c
        pltpu.sync_copy(src_hbm.at[wid], srcv)
        pltpu.sync_copy(dst_hbm.at[wid], dstv)
        pltpu.sync_copy(ew_hbm.at[wid], ewv)
        pltpu.sync_copy(zero_hbm, acc.at[pl.ds(s * _RPT, _RPT)])
        if table_in_spmem:
            pltpu.sync_copy(h_hbm.at[pl.ds(s * _RPT, _RPT)],
                            tab[0].at[pl.ds(s * _RPT, _RPT)])
        plsc.subcore_barrier()
        gather_src = tab[0] if table_in_spmem else h_hbm

        def scale(j):
            def grp_body(g, carry2):
                ewg = ewv[j, pl.ds(g * 16, 16)]
                for l in range(16):
                    wvec = lax.gather(
                        ewg, jnp.full((16, 1), l, jnp.int32),
                        lax.GatherDimensionNumbers(
                            offset_dims=(), collapsed_slice_dims=(0,),
                            start_index_map=(0,)),
                        slice_sizes=(1,),
                        mode=lax.GatherScatterMode.PROMISE_IN_BOUNDS)
                    r = g * 16 + l
                    for f in range(D // 16):
                        sl = pl.ds(f * 16, 16)
                        rows[r, sl] = rows[r, sl] * wvec
                return carry2

            lax.fori_loop(0, _CHUNK // 16, grp_body, 0)

        def chunk_body(j, carry):
            pltpu.async_copy(gather_src.at[srcv.at[j]], rows, sem).wait()
            scale(j)
            pltpu.sync_copy(rows, acc.at[dstv.at[j]], add=True)
            return carry

        lax.fori_loop(0, _CPT, chunk_body, 0)
        plsc.subcore_barrier()
        pltpu.sync_copy(acc.at[pl.ds(s * _RPT, _RPT)],
                        out_hbm.at[c, pl.ds(s * _RPT, _RPT)])

    return sc_pass


_sc_pass_128 = _make_sc_pass(_F)
_sc_pass_64 = _make_sc_pass(_DPAD, table_in_spmem=True)


def _tc_matmul(x, w):
    def body(x_ref, w_ref, o_ref):
        o_ref[...] = jnp.dot(x_ref[...], w_ref[...],
                             preferred_element_type=jnp.float32)

    return pl.pallas_call(
        body,
        out_shape=jax.ShapeDtypeStruct((x.shape[0], w.shape[1]), jnp.float32),
    )(x, w)


def _tc_mid(parts, b1, w2p):
    def body(p_ref, b1_ref, w_ref, o_ref):
        t = p_ref[0, :_N] + p_ref[1, :_N] + b1_ref[...]
        cm = jnp.mean(t, axis=0, keepdims=True)
        xc = t - cm
        ms = jnp.sum(xc * xc) / _N
        inv = lax.rsqrt(ms + 1e-6)
        h = jnp.maximum(xc * inv, 0.0)
        o_ref[...] = jnp.dot(h, w_ref[...], preferred_element_type=jnp.float32)

    return pl.pallas_call(
        body,
        out_shape=jax.ShapeDtypeStruct((_N, _DPAD), jnp.float32),
    )(parts, b1.reshape(1, -1), w2p)


def _tc_final(parts, b2):
    def body(q_ref, b2_ref, o_ref):
        ssum = q_ref[0, :_N] + q_ref[1, :_N]
        o_ref[...] = ssum[:, :_NCLASS] + b2_ref[...]

    return pl.pallas_call(
        body,
        out_shape=jax.ShapeDtypeStruct((_N, _NCLASS), jnp.float32),
    )(parts, b2.reshape(1, -1))


def kernel(x, edge_index, edge_attr, W1, b1, W2, b2):
    src = edge_index[0].astype(jnp.int32)
    dst = edge_index[1].astype(jnp.int32)
    ew = edge_attr.astype(jnp.float32)
    pad = _EPAD - _E
    src2 = jnp.concatenate([src, jnp.zeros((pad,), jnp.int32)]
                           ).reshape(_NW, _CPT, _CHUNK)
    dst2 = jnp.concatenate([dst, jnp.zeros((pad,), jnp.int32)]
                           ).reshape(_NW, _CPT, _CHUNK)
    ew2 = jnp.concatenate([ew, jnp.zeros((pad,), jnp.float32)]
                          ).reshape(_NW, _CPT, _CHUNK)
    zeros_f = jnp.zeros((_RPT, _F), jnp.float32)
    zeros_d = jnp.zeros((_RPT, _DPAD), jnp.float32)
    w2p = jnp.pad(W2, ((0, 0), (0, _DPAD - _NCLASS)))

    h1 = _tc_matmul(x, W1)
    p1 = _sc_pass_128(h1, src2, dst2, ew2, zeros_f)
    p = _tc_mid(p1, b1, w2p)
    p_pad = jnp.pad(p, ((0, _NPAD - _N), (0, 0)))
    p2 = _sc_pass_64(p_pad, src2, dst2, ew2, zeros_d)
    return _tc_final(p2, b2)


# speedup vs baseline: 1.6755x; 1.1233x over previous
import functools

import jax
import jax.numpy as jnp
from jax import lax
from jax.experimental import pallas as pl
from jax.experimental.pallas import tpu as pltpu
from jax.experimental.pallas import tpu_sc as plsc

_N = 10000
_F = 128
_NCLASS = 40
_DPAD = 64
_E = 320000
_CHUNK = 128
_NC = 2
_NS = 16
_NW = _NC * _NS
_CPT = -(-_E // (_NW * _CHUNK))
_EPAD = _NW * _CHUNK * _CPT
_NPAD = 10240
_RPT = _NPAD // _NS


def _make_sc_pass(D, table_in_spmem=False):
    mesh = plsc.VectorSubcoreMesh(core_axis_name="c", subcore_axis_name="s")
    tab_scratch = ([pltpu.VMEM_SHARED((_NPAD, D), jnp.float32)]
                   if table_in_spmem else [])

    @functools.partial(
        pl.kernel,
        mesh=mesh,
        compiler_params=pltpu.CompilerParams(use_tc_tiling_on_sc=False),
        out_type=jax.ShapeDtypeStruct((_NC, _NPAD, D), jnp.float32),
        scratch_types=[
            pltpu.VMEM_SHARED((_NPAD, D), jnp.float32),
            pltpu.VMEM((_CPT, _CHUNK), jnp.int32),
            pltpu.VMEM((_CPT, _CHUNK), jnp.int32),
            pltpu.VMEM((_CPT, _CHUNK), jnp.float32),
            pltpu.VMEM((_CHUNK, D), jnp.float32),
            pltpu.SemaphoreType.DMA,
        ] + tab_scratch,
    )
    def sc_pass(h_hbm, src_hbm, dst_hbm, ew_hbm, zero_hbm, out_hbm,
                acc, srcv, dstv, ewv, rows, sem, *tab):
        c = lax.axis_index("c")
        s = lax.axis_index("s")
        wid = s * _NC + c
        pltpu.sync_copy(src_hbm.at[wid], srcv)
        pltpu.sync_copy(dst_hbm.at[wid], dstv)
        pltpu.sync_copy(ew_hbm.at[wid], ewv)
        pltpu.sync_copy(zero_hbm, acc.at[pl.ds(s * _RPT, _RPT)])
        if table_in_spmem:
            pltpu.sync_copy(h_hbm.at[pl.ds(s * _RPT, _RPT)],
                            tab[0].at[pl.ds(s * _RPT, _RPT)])
        plsc.subcore_barrier()
        gather_src = tab[0] if table_in_spmem else h_hbm

        def scale(j):
            def grp_body(g, carry2):
                ewg = ewv[j, pl.ds(g * 16, 16)]
                for l in range(16):
                    wvec = lax.gather(
                        ewg, jnp.full((16, 1), l, jnp.int32),
                        lax.GatherDimensionNumbers(
                            offset_dims=(), collapsed_slice_dims=(0,),
                            start_index_map=(0,)),
                        slice_sizes=(1,),
                        mode=lax.GatherScatterMode.PROMISE_IN_BOUNDS)
                    r = g * 16 + l
                    for f in range(D // 16):
                        sl = pl.ds(f * 16, 16)
                        rows[r, sl] = rows[r, sl] * wvec
                return carry2

            lax.fori_loop(0, _CHUNK // 16, grp_body, 0)

        def chunk_body(j, carry):
            pltpu.async_copy(gather_src.at[srcv.at[j]], rows, sem).wait()
            scale(j)
            pltpu.sync_copy(rows, acc.at[dstv.at[j]], add=True)
            return carry

        lax.fori_loop(0, _CPT, chunk_body, 0)
        plsc.subcore_barrier()
        pltpu.sync_copy(acc.at[pl.ds(s * _RPT, _RPT)],
                        out_hbm.at[c, pl.ds(s * _RPT, _RPT)])

    return sc_pass


_sc_pass_128 = _make_sc_pass(_F)
_sc_pass_64 = _make_sc_pass(_DPAD)


def _tc_matmul(x, w):
    def body(x_ref, w_ref, o_ref):
        o_ref[...] = jnp.dot(x_ref[...], w_ref[...],
                             preferred_element_type=jnp.float32)

    return pl.pallas_call(
        body,
        out_shape=jax.ShapeDtypeStruct((x.shape[0], w.shape[1]), jnp.float32),
    )(x, w)


def _tc_mid(parts, b1, w2p):
    def body(p_ref, b1_ref, w_ref, o_ref):
        t = p_ref[0, :_N] + p_ref[1, :_N] + b1_ref[...]
        cm = jnp.mean(t, axis=0, keepdims=True)
        xc = t - cm
        ms = jnp.sum(xc * xc) / _N
        inv = lax.rsqrt(ms + 1e-6)
        h = jnp.maximum(xc * inv, 0.0)
        o_ref[...] = jnp.dot(h, w_ref[...], preferred_element_type=jnp.float32)

    return pl.pallas_call(
        body,
        out_shape=jax.ShapeDtypeStruct((_N, _DPAD), jnp.float32),
    )(parts, b1.reshape(1, -1), w2p)


def _tc_final(parts, b2):
    def body(q_ref, b2_ref, o_ref):
        ssum = q_ref[0, :_N] + q_ref[1, :_N]
        o_ref[...] = ssum[:, :_NCLASS] + b2_ref[...]

    return pl.pallas_call(
        body,
        out_shape=jax.ShapeDtypeStruct((_N, _NCLASS), jnp.float32),
    )(parts, b2.reshape(1, -1))


def kernel(x, edge_index, edge_attr, W1, b1, W2, b2):
    src = edge_index[0].astype(jnp.int32)
    dst = edge_index[1].astype(jnp.int32)
    ew = edge_attr.astype(jnp.float32)
    pad = _EPAD - _E
    src2 = jnp.concatenate([src, jnp.zeros((pad,), jnp.int32)]
                           ).reshape(_NW, _CPT, _CHUNK)
    dst2 = jnp.concatenate([dst, jnp.zeros((pad,), jnp.int32)]
                           ).reshape(_NW, _CPT, _CHUNK)
    ew2 = jnp.concatenate([ew, jnp.zeros((pad,), jnp.float32)]
                          ).reshape(_NW, _CPT, _CHUNK)
    zeros_f = jnp.zeros((_RPT, _F), jnp.float32)
    zeros_d = jnp.zeros((_RPT, _DPAD), jnp.float32)
    w2p = jnp.pad(W2, ((0, 0), (0, _DPAD - _NCLASS)))

    h1 = _tc_matmul(x, W1)
    p1 = _sc_pass_128(h1, src2, dst2, ew2, zeros_f)
    p = _tc_mid(p1, b1, w2p)
    p2 = _sc_pass_64(p, src2, dst2, ew2, zeros_d)
    return _tc_final(p2, b2)
